# baseline (speedup 1.0000x reference)
import functools
import jax
import jax.numpy as jnp
from jax import lax
from jax.experimental import pallas as pl
from jax.experimental.pallas import tpu as pltpu

NN = 100000
BB = 128
HH = 32
BLK = 5000
NG = NN // BLK


def _leaky(t):
    return jnp.where(t > 0.0, t, 0.2 * t)


def _dense_body(x_ref, w_ref, asv_ref, adv_ref, h_ref, as_ref, ad_ref, mm_ref, c_ref):
    g = pl.program_id(0)
    h = jnp.dot(x_ref[...], w_ref[...], preferred_element_type=jnp.float32)
    h_ref[...] = h
    a_s = jnp.dot(h, asv_ref[...], preferred_element_type=jnp.float32)
    a_d = jnp.dot(h, adv_ref[...], preferred_element_type=jnp.float32)
    as_ref[...] = a_s
    ad_ref[...] = a_d
    cur = jnp.concatenate([jnp.max(a_s).reshape(1, 1), jnp.max(a_d).reshape(1, 1)], axis=1)

    @pl.when(g == 0)
    def _():
        mm_ref[...] = cur

    @pl.when(g > 0)
    def _():
        mm_ref[...] = jnp.maximum(mm_ref[...], cur)

    @pl.when(g == NG - 1)
    def _():
        m = mm_ref[0, 0] + mm_ref[0, 1]
        c_ref[...] = jnp.zeros((1, 1), jnp.float32) + _leaky(m)


def _dense(x, W, a_src, a_dst):
    in_dim = x.shape[1]
    h, a_s, a_d, _, c = pl.pallas_call(
        _dense_body,
        grid=(NG,),
        in_specs=[
            pl.BlockSpec((BLK, in_dim), lambda g: (g, 0)),
            pl.BlockSpec((in_dim, HH), lambda g: (0, 0)),
            pl.BlockSpec((HH, 1), lambda g: (0, 0)),
            pl.BlockSpec((HH, 1), lambda g: (0, 0)),
        ],
        out_specs=(
            pl.BlockSpec((BLK, HH), lambda g: (g, 0)),
            pl.BlockSpec((BLK, 1), lambda g: (g, 0)),
            pl.BlockSpec((BLK, 1), lambda g: (g, 0)),
            pl.BlockSpec((1, 2), lambda g: (0, 0)),
            pl.BlockSpec((1, 1), lambda g: (0, 0)),
        ),
        out_shape=(
            jax.ShapeDtypeStruct((NN, HH), jnp.float32),
            jax.ShapeDtypeStruct((NN, 1), jnp.float32),
            jax.ShapeDtypeStruct((NN, 1), jnp.float32),
            jax.ShapeDtypeStruct((1, 2), jnp.float32),
            jax.ShapeDtypeStruct((1, 1), jnp.float32),
        ),
    )(x, W, a_src[:, None], a_dst[:, None])
    return h, a_s, a_d, c


def _combine_body(acc_ref, den_ref, h_ref, as_ref, ad_ref, c_ref, b_ref, o_ref, *, relu):
    t = as_ref[...] + ad_ref[...]
    ws = jnp.exp(_leaky(t) - c_ref[0, 0])
    num = acc_ref[...] + h_ref[...] * ws
    den = den_ref[...] + ws
    o = num / den + b_ref[...]
    if relu:
        o = jnp.maximum(o, 0.0)
    o_ref[...] = o


def _combine(acc, den, h, a_s, a_d, c, b, relu):
    return pl.pallas_call(
        functools.partial(_combine_body, relu=relu),
        grid=(NG,),
        in_specs=[
            pl.BlockSpec((BLK, HH), lambda g: (g, 0)),
            pl.BlockSpec((BLK, 1), lambda g: (g, 0)),
            pl.BlockSpec((BLK, HH), lambda g: (g, 0)),
            pl.BlockSpec((BLK, 1), lambda g: (g, 0)),
            pl.BlockSpec((BLK, 1), lambda g: (g, 0)),
            pl.BlockSpec((1, 1), lambda g: (0, 0)),
            pl.BlockSpec((1, HH), lambda g: (0, 0)),
        ],
        out_specs=pl.BlockSpec((BLK, HH), lambda g: (g, 0)),
        out_shape=jax.ShapeDtypeStruct((NN, HH), jnp.float32),
    )(acc, den, h, a_s, a_d, c, b[None, :])


def _pool_body(h_ref, batch_ref, wl_ref, bl_ref, o_ref, s_ref, cnt_ref):
    g = pl.program_id(0)

    @pl.when(g == 0)
    def _():
        s_ref[...] = jnp.zeros_like(s_ref)
        cnt_ref[...] = jnp.zeros_like(cnt_ref)

    oh = (batch_ref[...] == lax.broadcasted_iota(jnp.int32, (1, BB), 1)).astype(jnp.float32)
    s_ref[...] += lax.dot_general(oh, h_ref[...], (((0,), (0,)), ((), ())),
                                  preferred_element_type=jnp.float32)
    cnt_ref[...] += jnp.sum(oh, axis=0)[:, None]

    @pl.when(g == NG - 1)
    def _():
        gmean = s_ref[...] / jnp.maximum(cnt_ref[...], 1.0)
        o_ref[...] = jnp.dot(gmean, wl_ref[...], preferred_element_type=jnp.float32) + bl_ref[...]


def _pool(h, batch, Wlin, blin):
    od = Wlin.shape[1]
    return pl.pallas_call(
        _pool_body,
        grid=(NG,),
        in_specs=[
            pl.BlockSpec((BLK, HH), lambda g: (g, 0)),
            pl.BlockSpec((BLK, 1), lambda g: (g, 0)),
            pl.BlockSpec((HH, od), lambda g: (0, 0)),
            pl.BlockSpec((1, od), lambda g: (0, 0)),
        ],
        out_specs=pl.BlockSpec((BB, od), lambda g: (0, 0)),
        out_shape=jax.ShapeDtypeStruct((BB, od), jnp.float32),
        scratch_shapes=[
            pltpu.VMEM((BB, HH), jnp.float32),
            pltpu.VMEM((BB, 1), jnp.float32),
        ],
    )(h, batch[:, None], Wlin, blin[None, :])


def _edges(a_s, a_d, c, h, src, dst):
    t = a_s[src, 0] + a_d[dst, 0]
    w = jnp.exp(_leaky(t) - c[0, 0])
    den = jax.ops.segment_sum(w, dst, num_segments=NN)
    acc = jax.ops.segment_sum(h[src] * w[:, None], dst, num_segments=NN)
    return acc, den[:, None]


def kernel(x, edge_index, batch, W1, a_src1, a_dst1, b1, W2, a_src2, a_dst2, b2,
           W3, a_src3, a_dst3, b3, Wlin, blin):
    src = edge_index[0]
    dst = edge_index[1]
    h_in = x
    layers = [
        (W1, a_src1, a_dst1, b1, True),
        (W2, a_src2, a_dst2, b2, True),
        (W3, a_src3, a_dst3, b3, False),
    ]
    for W, asv, adv, b, relu in layers:
        h, a_s, a_d, c = _dense(h_in, W, asv, adv)
        acc, den = _edges(a_s, a_d, c, h, src, dst)
        h_in = _combine(acc, den, h, a_s, a_d, c, b, relu)
    return _pool(h_in, batch, Wlin, blin)


# speedup vs baseline: 16.3546x; 16.3546x over previous
import functools
import jax
import jax.numpy as jnp
from jax import lax
from jax.experimental import pallas as pl
from jax.experimental.pallas import tpu as pltpu
from jax.experimental.pallas import tpu_sc as plsc

NN = 100000
BB = 128
HH = 32
BLK = 5000
NG = NN // BLK

EE = 1600000
NSUB = 16
EPS = EE // NSUB
KCH = 400
NCH = EPS // KCH
QTR = NN // 4
QP = QTR + 88
ZROW = QP // NSUB


def _leaky(t):
    return jnp.where(t > 0.0, t, 0.2 * t)


def _dense_body(x_ref, w_ref, asv_ref, adv_ref, h_ref, as_ref, ad_ref, mm_ref, c_ref):
    g = pl.program_id(0)
    h = jnp.dot(x_ref[...], w_ref[...], preferred_element_type=jnp.float32)
    h_ref[...] = h
    a_s = jnp.dot(h, asv_ref[...], preferred_element_type=jnp.float32)
    a_d = jnp.dot(h, adv_ref[...], preferred_element_type=jnp.float32)
    as_ref[...] = a_s
    ad_ref[...] = a_d
    cur = jnp.concatenate([jnp.max(a_s).reshape(1, 1), jnp.max(a_d).reshape(1, 1)], axis=1)

    @pl.when(g == 0)
    def _():
        mm_ref[...] = cur

    @pl.when(g > 0)
    def _():
        mm_ref[...] = jnp.maximum(mm_ref[...], cur)

    @pl.when(g == NG - 1)
    def _():
        m = mm_ref[0, 0] + mm_ref[0, 1]
        c_ref[...] = jnp.zeros((1, 1), jnp.float32) + _leaky(m)


def _dense(x, W, a_src, a_dst):
    in_dim = x.shape[1]
    h, a_s, a_d, _, c = pl.pallas_call(
        _dense_body,
        grid=(NG,),
        in_specs=[
            pl.BlockSpec((BLK, in_dim), lambda g: (g, 0)),
            pl.BlockSpec((in_dim, HH), lambda g: (0, 0)),
            pl.BlockSpec((HH, 1), lambda g: (0, 0)),
            pl.BlockSpec((HH, 1), lambda g: (0, 0)),
        ],
        out_specs=(
            pl.BlockSpec((BLK, HH), lambda g: (g, 0)),
            pl.BlockSpec((BLK, 1), lambda g: (g, 0)),
            pl.BlockSpec((BLK, 1), lambda g: (g, 0)),
            pl.BlockSpec((1, 2), lambda g: (0, 0)),
            pl.BlockSpec((1, 1), lambda g: (0, 0)),
        ),
        out_shape=(
            jax.ShapeDtypeStruct((NN, HH), jnp.float32),
            jax.ShapeDtypeStruct((NN, 1), jnp.float32),
            jax.ShapeDtypeStruct((NN, 1), jnp.float32),
            jax.ShapeDtypeStruct((1, 2), jnp.float32),
            jax.ShapeDtypeStruct((1, 1), jnp.float32),
        ),
    )(x, W, a_src[:, None], a_dst[:, None])
    return h, a_s, a_d, c


def _combine_body(acc_ref, den_ref, h_ref, as_ref, ad_ref, c_ref, b_ref, o_ref, *, relu):
    t = as_ref[...] + ad_ref[...]
    ws = jnp.exp(_leaky(t) - c_ref[0, 0])
    num = acc_ref[...] + h_ref[...] * ws
    den = den_ref[...] + ws
    o = num / den + b_ref[...]
    if relu:
        o = jnp.maximum(o, 0.0)
    o_ref[...] = o


def _combine(acc, den, h, a_s, a_d, c, b, relu):
    return pl.pallas_call(
        functools.partial(_combine_body, relu=relu),
        grid=(NG,),
        in_specs=[
            pl.BlockSpec((BLK, HH), lambda g: (g, 0)),
            pl.BlockSpec((BLK, 1), lambda g: (g, 0)),
            pl.BlockSpec((BLK, HH), lambda g: (g, 0)),
            pl.BlockSpec((BLK, 1), lambda g: (g, 0)),
            pl.BlockSpec((BLK, 1), lambda g: (g, 0)),
            pl.BlockSpec((1, 1), lambda g: (0, 0)),
            pl.BlockSpec((1, HH), lambda g: (0, 0)),
        ],
        out_specs=pl.BlockSpec((BLK, HH), lambda g: (g, 0)),
        out_shape=jax.ShapeDtypeStruct((NN, HH), jnp.float32),
    )(acc, den, h, a_s, a_d, c, b[None, :])


def _pool_body(h_ref, batch_ref, wl_ref, bl_ref, o_ref, s_ref, cnt_ref):
    g = pl.program_id(0)

    @pl.when(g == 0)
    def _():
        s_ref[...] = jnp.zeros_like(s_ref)
        cnt_ref[...] = jnp.zeros_like(cnt_ref)

    oh = (batch_ref[...] == lax.broadcasted_iota(jnp.int32, (1, BB), 1)).astype(jnp.float32)
    s_ref[...] += lax.dot_general(oh, h_ref[...], (((0,), (0,)), ((), ())),
                                  preferred_element_type=jnp.float32)
    cnt_ref[...] += jnp.sum(oh, axis=0)[:, None]

    @pl.when(g == NG - 1)
    def _():
        gmean = s_ref[...] / jnp.maximum(cnt_ref[...], 1.0)
        o_ref[...] = jnp.dot(gmean, wl_ref[...], preferred_element_type=jnp.float32) + bl_ref[...]


def _pool(h, batch, Wlin, blin):
    od = Wlin.shape[1]
    return pl.pallas_call(
        _pool_body,
        grid=(NG,),
        in_specs=[
            pl.BlockSpec((BLK, HH), lambda g: (g, 0)),
            pl.BlockSpec((BLK, 1), lambda g: (g, 0)),
            pl.BlockSpec((HH, od), lambda g: (0, 0)),
            pl.BlockSpec((1, od), lambda g: (0, 0)),
        ],
        out_specs=pl.BlockSpec((BB, od), lambda g: (0, 0)),
        out_shape=jax.ShapeDtypeStruct((BB, od), jnp.float32),
        scratch_shapes=[
            pltpu.VMEM((BB, HH), jnp.float32),
            pltpu.VMEM((BB, 1), jnp.float32),
        ],
    )(h, batch[:, None], Wlin, blin[None, :])


def _edge_body(src_hbm, dst_hbm, as8_hbm, ad8_hbm, h_hbm, cvec_hbm, zacc_hbm,
               acc_hbm, srcbuf, dstbuf, as_v, ad_v, rows, rows40, idx, cbuf,
               acc_sh, sem0, sem1, sem2, *, tq):
    ci = lax.axis_index("c")
    si = lax.axis_index("s")
    iota = lax.iota(jnp.int32, 16)
    zeros16 = jnp.zeros((16,), jnp.int32)
    zerosf = jnp.zeros((16,), jnp.float32)

    pltpu.sync_copy(zacc_hbm.at[pl.ds(si * ZROW, ZROW)], acc_sh.at[pl.ds(si * ZROW, ZROW)])
    pltpu.sync_copy(cvec_hbm, cbuf)

    for cc in range(33, 40):
        def zb(g, carry):
            plsc.store_scatter(rows40, [g * 16 + iota, zeros16 + cc], zerosf)
            return carry
        lax.fori_loop(0, KCH // 16, zb, 0)

    plsc.subcore_barrier()

    cv = cbuf[...]
    lo = (2 * tq + ci) * QTR

    def chunk(j, carry):
        b = si * EPS + j * KCH
        pltpu.sync_copy(src_hbm.at[pl.ds(b, KCH)], srcbuf)
        pltpu.sync_copy(dst_hbm.at[pl.ds(b, KCH)], dstbuf)
        ca = pltpu.async_copy(as8_hbm.at[srcbuf], as_v, sem0)
        cb = pltpu.async_copy(ad8_hbm.at[dstbuf], ad_v, sem1)
        cc2 = pltpu.async_copy(h_hbm.at[srcbuf], rows, sem2)
        ca.wait()
        cb.wait()
        cc2.wait()

        def grp(g, carry2):
            ids = g * 16 + iota
            a_s = plsc.load_gather(as_v, [ids, zeros16])
            a_d = plsc.load_gather(ad_v, [ids, zeros16])
            t = a_s + a_d
            w = jnp.exp(jnp.where(t > 0.0, t, 0.2 * t) - cv)
            plsc.store_scatter(rows40, [ids, zeros16 + 32], w)
            d = dstbuf[pl.ds(g * 16, 16)]
            ld = d - lo
            valid = (ld >= 0) & (ld < QTR)
            idx[pl.ds(g * 16, 16)] = jnp.where(valid, ld, QTR)
            return carry2
        lax.fori_loop(0, KCH // 16, grp, 0)

        def scale(e, carry2):
            erep = zeros16 + e
            wb = plsc.load_gather(rows40, [erep, zeros16 + 32])
            c0 = iota
            c1 = iota + 16
            r0 = plsc.load_gather(rows, [erep, c0])
            r1 = plsc.load_gather(rows, [erep, c1])
            plsc.store_scatter(rows40, [erep, c0], r0 * wb)
            plsc.store_scatter(rows40, [erep, c1], r1 * wb)
            return carry2
        lax.fori_loop(0, KCH, scale, 0)

        pltpu.sync_copy(rows40, acc_sh.at[idx], add=True)
        return carry
    lax.fori_loop(0, NCH, chunk, 0)

    plsc.subcore_barrier()

    r0 = si * ZROW
    base = ci * QP
    pltpu.sync_copy(acc_sh.at[pl.ds(r0, ZROW)], acc_hbm.at[pl.ds(base + r0, ZROW)])


def _edges(a_s, a_d, c, h, src, dst):
    as8 = jnp.broadcast_to(a_s, (NN, 8))
    ad8 = jnp.broadcast_to(a_d, (NN, 8))
    cvec = jnp.broadcast_to(c.reshape(1), (16,))
    zacc = jnp.zeros((QP, 40), jnp.float32)
    mesh = plsc.VectorSubcoreMesh(core_axis_name="c", subcore_axis_name="s")
    outs = []
    for tq in (0, 1):
        o = pl.kernel(
            functools.partial(_edge_body, tq=tq),
            mesh=mesh,
            compiler_params=pltpu.CompilerParams(use_tc_tiling_on_sc=False,
                                                 needs_layout_passes=False),
            out_type=jax.ShapeDtypeStruct((2 * QP, 40), jnp.float32),
            scratch_types=[
                pltpu.VMEM((KCH,), jnp.int32),
                pltpu.VMEM((KCH,), jnp.int32),
                pltpu.VMEM((KCH, 8), jnp.float32),
                pltpu.VMEM((KCH, 8), jnp.float32),
                pltpu.VMEM((KCH, HH), jnp.float32),
                pltpu.VMEM((KCH, 40), jnp.float32),
                pltpu.VMEM((KCH,), jnp.int32),
                pltpu.VMEM((16,), jnp.float32),
                pltpu.VMEM_SHARED((QP, 40), jnp.float32),
                pltpu.SemaphoreType.DMA,
                pltpu.SemaphoreType.DMA,
                pltpu.SemaphoreType.DMA,
            ],
        )(src, dst, as8, ad8, h, cvec, zacc)
        outs.append(o)
    full = jnp.concatenate([outs[0][:QTR], outs[0][QP:QP + QTR],
                            outs[1][:QTR], outs[1][QP:QP + QTR]], axis=0)
    return full[:, :HH], full[:, 32:33]


def kernel(x, edge_index, batch, W1, a_src1, a_dst1, b1, W2, a_src2, a_dst2, b2,
           W3, a_src3, a_dst3, b3, Wlin, blin):
    src = edge_index[0]
    dst = edge_index[1]
    h_in = x
    layers = [
        (W1, a_src1, a_dst1, b1, True),
        (W2, a_src2, a_dst2, b2, True),
        (W3, a_src3, a_dst3, b3, False),
    ]
    for W, asv, adv, b, relu in layers:
        h, a_s, a_d, c = _dense(h_in, W, asv, adv)
        acc, den = _edges(a_s, a_d, c, h, src, dst)
        h_in = _combine(acc, den, h, a_s, a_d, c, b, relu)
    return _pool(h_in, batch, Wlin, blin)
